# baseline (speedup 1.0000x reference)
import functools

import jax
import jax.numpy as jnp
from jax import lax
from jax.experimental import pallas as pl
from jax.experimental.pallas import tpu as pltpu
from jax.experimental.pallas import tpu_sc as plsc

D = 64
B = 16384
NNEG = 20
NC = 2
NS = 16
NW = NC * NS
BPW = B // NW
CHUNK = 256
UGRP = 16


_UW = 4


def _sc_gather_u(ut, pos_u):
    mesh = plsc.VectorSubcoreMesh(core_axis_name="c", subcore_axis_name="s")

    @functools.partial(
        pl.kernel,
        out_type=jax.ShapeDtypeStruct((B, D), jnp.float32),
        mesh=mesh,
        compiler_params=pltpu.CompilerParams(use_tc_tiling_on_sc=True,
                                             needs_layout_passes=False),
        scratch_types=[
            pltpu.VMEM((BPW,), jnp.int32),
        ] + [pltpu.VMEM((D, 128), jnp.float32)] * _UW + [
            pltpu.VMEM((BPW, D), jnp.float32),
            pltpu.SemaphoreType.DMA,
            pltpu.SemaphoreType.DMA,
        ],
    )
    def gather_u_kernel(ut_hbm, pu_hbm, eu_hbm, idx_v, st0, st1, st2, st3,
                        rows_v, sem_i, sem_r):
        sts = [st0, st1, st2, st3]
        wid = lax.axis_index("s") * NC + lax.axis_index("c")
        base = wid * BPW
        pltpu.async_copy(pu_hbm.at[pl.ds(base, BPW)], idx_v, sem_i).wait()
        lanes = lax.iota(jnp.int32, 16)

        @pl.loop(0, BPW, step=16)
        def _(j):
            v16 = idx_v[pl.ds(j, 16)]
            for w in range(16 // _UW):
                for k in range(_UW):
                    c0 = pl.multiple_of((v16[w * _UW + k] >> 7) << 7, 128)
                    pltpu.async_copy(ut_hbm.at[pl.ds(0, D), pl.ds(c0, 128)],
                                     sts[k], sem_r)
                for k in range(_UW):
                    pltpu.make_async_copy(ut_hbm.at[pl.ds(0, D),
                                                    pl.ds(0, 128)],
                                          sts[k], sem_r).wait()
                for k in range(_UW):
                    i = v16[w * _UW + k]
                    col = jnp.broadcast_to(i & 127, (16,))
                    row = jnp.broadcast_to(j + w * _UW + k, (16,))
                    for q in range(D // 16):
                        vals = plsc.load_gather(
                            sts[k], [16 * q + lanes, col])
                        plsc.store_scatter(
                            rows_v, [row, 16 * q + lanes], vals)

        pltpu.sync_copy(rows_v, eu_hbm.at[pl.ds(base, BPW)])

    return gather_u_kernel(ut, pos_u)


def _sc_gather_v(v_weight, pos_v, neg_v_flat):
    mesh = plsc.VectorSubcoreMesh(core_axis_name="c", subcore_axis_name="s")

    @functools.partial(
        pl.kernel,
        out_type=[
            jax.ShapeDtypeStruct((B, D), jnp.float32),
            jax.ShapeDtypeStruct((B * NNEG, D), jnp.float32),
        ],
        mesh=mesh,
        compiler_params=pltpu.CompilerParams(use_tc_tiling_on_sc=True),
        scratch_types=[
            pltpu.VMEM((CHUNK,), jnp.int32),
            pltpu.VMEM((CHUNK, D), jnp.float32),
            pltpu.SemaphoreType.DMA,
            pltpu.SemaphoreType.DMA,
        ],
    )
    def gather_v_kernel(v_hbm, pv_hbm, nv_hbm, ev_hbm, en_hbm,
                        idx_v, rows_v, sem_i, sem_r):
        wid = lax.axis_index("s") * NC + lax.axis_index("c")

        def stream(idx_hbm, out_hbm, base, nrows):
            @pl.loop(0, nrows // CHUNK)
            def _(i):
                off = base + i * CHUNK
                pltpu.async_copy(idx_hbm.at[pl.ds(off, CHUNK)], idx_v,
                                 sem_i).wait()

                @pl.loop(0, CHUNK, step=16)
                def _(j):
                    v16 = idx_v[pl.ds(j, 16)]
                    for k in range(16):
                        pltpu.async_copy(v_hbm.at[pl.ds(v16[k], 1)],
                                         rows_v.at[pl.ds(j + k, 1)], sem_r)

                pltpu.make_async_copy(
                    v_hbm.at[pl.ds(0, CHUNK)], rows_v, sem_r).wait()
                pltpu.sync_copy(rows_v, out_hbm.at[pl.ds(off, CHUNK)])

        stream(pv_hbm, ev_hbm, wid * BPW, BPW)
        stream(nv_hbm, en_hbm, wid * BPW * NNEG, BPW * NNEG)

    return gather_v_kernel(v_weight, pos_v, neg_v_flat)


_TC_BLK = 512
_NBLK = B // _TC_BLK


def _tc_body(eu_ref, ev_ref, en_ref, out_ref):
    u = eu_ref[...]
    v = ev_ref[...]
    neg = en_ref[...]

    LOG2E = 1.4426950408889634
    LN2 = 0.6931471805599453

    def nlogsig(sx):
        return jnp.log2(1.0 + jnp.exp2(sx * LOG2E)) * LN2

    ones = jnp.ones((D, 128), jnp.float32)
    prod3 = neg.reshape(_TC_BLK, NNEG, D) * u[:, None, :]
    pos_score = jax.lax.dot(u * v, ones)[:, :1]
    neg_score = jax.lax.dot(prod3.reshape(_TC_BLK * NNEG, D), ones)[:, :1]
    total = jnp.sum(nlogsig(-pos_score)) + jnp.sum(nlogsig(neg_score))
    out_ref[...] = jnp.full((1, 1, 128), total, jnp.float32)


def _tc_loss(emb_u, emb_v, neg_rows):
    partials = pl.pallas_call(
        _tc_body,
        grid=(_NBLK,),
        in_specs=[
            pl.BlockSpec((_TC_BLK, D), lambda i: (i, 0)),
            pl.BlockSpec((_TC_BLK, D), lambda i: (i, 0)),
            pl.BlockSpec((_TC_BLK * NNEG, D), lambda i: (i, 0)),
        ],
        out_specs=pl.BlockSpec((1, 1, 128), lambda i: (i, 0, 0)),
        out_shape=jax.ShapeDtypeStruct((_NBLK, 1, 128), jnp.float32),
        compiler_params=pltpu.CompilerParams(
            dimension_semantics=("parallel",)),
    )(emb_u, emb_v, neg_rows)
    return jnp.sum(partials[:, 0, 0])


def kernel(u_weight, v_weight, pos_u, pos_v, neg_v):
    neg_flat = neg_v.reshape(B * NNEG)
    emb_u = _sc_gather_u(u_weight, pos_u.astype(jnp.int32))
    emb_v, neg_rows = _sc_gather_v(
        v_weight, pos_v.astype(jnp.int32), neg_flat.astype(jnp.int32))
    return _tc_loss(emb_u, emb_v, neg_rows)


# speedup vs baseline: 1.3089x; 1.3089x over previous
import functools

import jax
import jax.numpy as jnp
from jax import lax
from jax.experimental import pallas as pl
from jax.experimental.pallas import tpu as pltpu
from jax.experimental.pallas import tpu_sc as plsc

D = 64
B = 16384
NNEG = 20
NC = 2
NS = 16
NW = NC * NS
BPW = B // NW
CHUNK = 256
UGRP = 16


def _sc_gather_u(u_weight, pos_u):
    mesh = plsc.VectorSubcoreMesh(core_axis_name="c", subcore_axis_name="s")

    @functools.partial(
        pl.kernel,
        out_type=jax.ShapeDtypeStruct((B, D), jnp.float32),
        mesh=mesh,
        compiler_params=pltpu.CompilerParams(use_tc_tiling_on_sc=True),
        scratch_types=[
            pltpu.VMEM((BPW,), jnp.int32),
            pltpu.VMEM((BPW, D), jnp.float32),
            pltpu.SemaphoreType.DMA,
            pltpu.SemaphoreType.DMA,
        ],
    )
    def gather_u_kernel(u_hbm, pu_hbm, eu_hbm, idx_v, rows_v, sem_i, sem_r):
        wid = lax.axis_index("s") * NC + lax.axis_index("c")
        base = wid * BPW
        pltpu.async_copy(pu_hbm.at[pl.ds(base, BPW)], idx_v, sem_i).wait()

        @pl.loop(0, BPW, step=16)
        def _(j):
            v16 = idx_v[pl.ds(j, 16)]
            for k in range(16):
                pltpu.async_copy(u_hbm.at[pl.ds(v16[k], 1)],
                                 rows_v.at[pl.ds(j + k, 1)], sem_r)

        pltpu.make_async_copy(u_hbm.at[pl.ds(0, BPW)], rows_v, sem_r).wait()
        pltpu.sync_copy(rows_v, eu_hbm.at[pl.ds(base, BPW)])

    return gather_u_kernel(u_weight, pos_u)


def _sc_gather_v(v_weight, pos_v, neg_v_flat):
    mesh = plsc.VectorSubcoreMesh(core_axis_name="c", subcore_axis_name="s")

    @functools.partial(
        pl.kernel,
        out_type=[
            jax.ShapeDtypeStruct((B, D), jnp.float32),
            jax.ShapeDtypeStruct((B * NNEG, D), jnp.float32),
        ],
        mesh=mesh,
        compiler_params=pltpu.CompilerParams(use_tc_tiling_on_sc=True),
        scratch_types=[
            pltpu.VMEM((CHUNK,), jnp.int32),
            pltpu.VMEM((CHUNK, D), jnp.float32),
            pltpu.SemaphoreType.DMA,
            pltpu.SemaphoreType.DMA,
        ],
    )
    def gather_v_kernel(v_hbm, pv_hbm, nv_hbm, ev_hbm, en_hbm,
                        idx_v, rows_v, sem_i, sem_r):
        wid = lax.axis_index("s") * NC + lax.axis_index("c")

        def stream(idx_hbm, out_hbm, base, nrows):
            @pl.loop(0, nrows // CHUNK)
            def _(i):
                off = base + i * CHUNK
                pltpu.async_copy(idx_hbm.at[pl.ds(off, CHUNK)], idx_v,
                                 sem_i).wait()

                @pl.loop(0, CHUNK, step=16)
                def _(j):
                    v16 = idx_v[pl.ds(j, 16)]
                    for k in range(16):
                        pltpu.async_copy(v_hbm.at[pl.ds(v16[k], 1)],
                                         rows_v.at[pl.ds(j + k, 1)], sem_r)

                pltpu.make_async_copy(
                    v_hbm.at[pl.ds(0, CHUNK)], rows_v, sem_r).wait()
                pltpu.sync_copy(rows_v, out_hbm.at[pl.ds(off, CHUNK)])

        stream(pv_hbm, ev_hbm, wid * BPW, BPW)
        stream(nv_hbm, en_hbm, wid * BPW * NNEG, BPW * NNEG)

    return gather_v_kernel(v_weight, pos_v, neg_v_flat)


_TC_BLK = 512
_NBLK = B // _TC_BLK


def _tc_body(eu_ref, ev_ref, en_ref, out_ref):
    u = eu_ref[...]
    v = ev_ref[...]
    neg = en_ref[...]

    LOG2E = 1.4426950408889634
    LN2 = 0.6931471805599453

    def nlogsig(sx):
        return jnp.log2(1.0 + jnp.exp2(sx * LOG2E)) * LN2

    ones = jnp.ones((D, 128), jnp.float32)
    prod3 = neg.reshape(_TC_BLK, NNEG, D) * u[:, None, :]
    pos_score = jax.lax.dot(u * v, ones)[:, :1]
    neg_score = jax.lax.dot(prod3.reshape(_TC_BLK * NNEG, D), ones)[:, :1]
    total = jnp.sum(nlogsig(-pos_score)) + jnp.sum(nlogsig(neg_score))
    out_ref[...] = jnp.full((1, 1, 128), total, jnp.float32)


def _tc_loss(emb_u, emb_v, neg_rows):
    partials = pl.pallas_call(
        _tc_body,
        grid=(_NBLK,),
        in_specs=[
            pl.BlockSpec((_TC_BLK, D), lambda i: (i, 0)),
            pl.BlockSpec((_TC_BLK, D), lambda i: (i, 0)),
            pl.BlockSpec((_TC_BLK * NNEG, D), lambda i: (i, 0)),
        ],
        out_specs=pl.BlockSpec((1, 1, 128), lambda i: (i, 0, 0)),
        out_shape=jax.ShapeDtypeStruct((_NBLK, 1, 128), jnp.float32),
        compiler_params=pltpu.CompilerParams(
            dimension_semantics=("parallel",)),
    )(emb_u, emb_v, neg_rows)
    return jnp.sum(partials[:, 0, 0])


def kernel(u_weight, v_weight, pos_u, pos_v, neg_v):
    neg_flat = neg_v.reshape(B * NNEG)
    emb_u = _sc_gather_u(u_weight, pos_u.astype(jnp.int32))
    emb_v, neg_rows = _sc_gather_v(
        v_weight, pos_v.astype(jnp.int32), neg_flat.astype(jnp.int32))
    return _tc_loss(emb_u, emb_v, neg_rows)


# speedup vs baseline: 1.3129x; 1.0031x over previous
import functools

import jax
import jax.numpy as jnp
from jax import lax
from jax.experimental import pallas as pl
from jax.experimental.pallas import tpu as pltpu
from jax.experimental.pallas import tpu_sc as plsc

D = 64
B = 16384
NNEG = 20
NC = 2
NS = 16
NW = NC * NS
BPW = B // NW
CHUNK = 256
UGRP = 16


def _sc_gather_u(u_weight, pos_u):
    mesh = plsc.VectorSubcoreMesh(core_axis_name="c", subcore_axis_name="s")

    @functools.partial(
        pl.kernel,
        out_type=jax.ShapeDtypeStruct((B, D), jnp.float32),
        mesh=mesh,
        compiler_params=pltpu.CompilerParams(use_tc_tiling_on_sc=True),
        scratch_types=[
            pltpu.VMEM((BPW,), jnp.int32),
            pltpu.VMEM((BPW, D), jnp.float32),
            pltpu.SemaphoreType.DMA,
            pltpu.SemaphoreType.DMA,
        ],
    )
    def gather_u_kernel(u_hbm, pu_hbm, eu_hbm, idx_v, rows_v, sem_i, sem_r):
        wid = lax.axis_index("s") * NC + lax.axis_index("c")
        base = wid * BPW
        pltpu.async_copy(pu_hbm.at[pl.ds(base, BPW)], idx_v, sem_i).wait()

        @pl.loop(0, BPW, step=16)
        def _(j):
            v16 = idx_v[pl.ds(j, 16)]
            for k in range(16):
                pltpu.async_copy(u_hbm.at[pl.ds(v16[k], 1)],
                                 rows_v.at[pl.ds(j + k, 1)], sem_r)

        pltpu.make_async_copy(u_hbm.at[pl.ds(0, BPW)], rows_v, sem_r).wait()
        pltpu.sync_copy(rows_v, eu_hbm.at[pl.ds(base, BPW)])

    return gather_u_kernel(u_weight, pos_u)


def _sc_gather_v(v_weight, pos_v, neg_v_flat):
    mesh = plsc.VectorSubcoreMesh(core_axis_name="c", subcore_axis_name="s")

    @functools.partial(
        pl.kernel,
        out_type=[
            jax.ShapeDtypeStruct((B, D), jnp.float32),
            jax.ShapeDtypeStruct((B * NNEG, D), jnp.float32),
        ],
        mesh=mesh,
        compiler_params=pltpu.CompilerParams(use_tc_tiling_on_sc=True),
        scratch_types=[
            pltpu.VMEM((CHUNK,), jnp.int32),
            pltpu.VMEM((CHUNK, D), jnp.float32),
            pltpu.SemaphoreType.DMA,
            pltpu.SemaphoreType.DMA,
        ],
    )
    def gather_v_kernel(v_hbm, pv_hbm, nv_hbm, ev_hbm, en_hbm,
                        idx_v, rows_v, sem_i, sem_r):
        wid = lax.axis_index("s") * NC + lax.axis_index("c")

        def stream(idx_hbm, out_hbm, base, nrows):
            @pl.loop(0, nrows // CHUNK)
            def _(i):
                off = base + i * CHUNK
                pltpu.async_copy(idx_hbm.at[pl.ds(off, CHUNK)], idx_v,
                                 sem_i).wait()

                @pl.loop(0, CHUNK, step=16)
                def _(j):
                    v16 = idx_v[pl.ds(j, 16)]
                    for k in range(16):
                        pltpu.async_copy(v_hbm.at[pl.ds(v16[k], 1)],
                                         rows_v.at[pl.ds(j + k, 1)], sem_r)

                pltpu.make_async_copy(
                    v_hbm.at[pl.ds(0, CHUNK)], rows_v, sem_r).wait()
                pltpu.sync_copy(rows_v, out_hbm.at[pl.ds(off, CHUNK)])

        stream(pv_hbm, ev_hbm, wid * BPW, BPW)
        stream(nv_hbm, en_hbm, wid * BPW * NNEG, BPW * NNEG)

    return gather_v_kernel(v_weight, pos_v, neg_v_flat)


_TC_BLK = 1024
_NBLK = B // _TC_BLK


def _tc_body(eu_ref, ev_ref, en_ref, out_ref):
    u = eu_ref[...]
    v = ev_ref[...]
    neg = en_ref[...]

    LOG2E = 1.4426950408889634
    LN2 = 0.6931471805599453

    def nlogsig(sx):
        return jnp.log2(1.0 + jnp.exp2(sx * LOG2E)) * LN2

    ones = jnp.ones((D, 128), jnp.float32)
    prod3 = neg.reshape(_TC_BLK, NNEG, D) * u[:, None, :]
    pos_score = jax.lax.dot(u * v, ones)[:, :1]
    neg_score = jax.lax.dot(prod3.reshape(_TC_BLK * NNEG, D), ones)[:, :1]
    total = jnp.sum(nlogsig(-pos_score)) + jnp.sum(nlogsig(neg_score))
    out_ref[...] = jnp.full((1, 1, 128), total, jnp.float32)


def _tc_loss(emb_u, emb_v, neg_rows):
    partials = pl.pallas_call(
        _tc_body,
        grid=(_NBLK,),
        in_specs=[
            pl.BlockSpec((_TC_BLK, D), lambda i: (i, 0)),
            pl.BlockSpec((_TC_BLK, D), lambda i: (i, 0)),
            pl.BlockSpec((_TC_BLK * NNEG, D), lambda i: (i, 0)),
        ],
        out_specs=pl.BlockSpec((1, 1, 128), lambda i: (i, 0, 0)),
        out_shape=jax.ShapeDtypeStruct((_NBLK, 1, 128), jnp.float32),
        compiler_params=pltpu.CompilerParams(
            dimension_semantics=("parallel",)),
    )(emb_u, emb_v, neg_rows)
    return jnp.sum(partials[:, 0, 0])


def kernel(u_weight, v_weight, pos_u, pos_v, neg_v):
    neg_flat = neg_v.reshape(B * NNEG)
    emb_u = _sc_gather_u(u_weight, pos_u.astype(jnp.int32))
    emb_v, neg_rows = _sc_gather_v(
        v_weight, pos_v.astype(jnp.int32), neg_flat.astype(jnp.int32))
    return _tc_loss(emb_u, emb_v, neg_rows)
